# baseline (speedup 1.0000x reference)
import functools

import jax
import jax.numpy as jnp
from jax import lax
from jax.experimental import pallas as pl
from jax.experimental.pallas import tpu as pltpu
from jax.experimental.pallas import tpu_sc as plsc

_B = 16384
_D = 64
_NW = 32
_BPW = _B // _NW
_CH = 64
_NCH = _BPW // _CH
_L = 16

_BS = 1024


def _gather_table(idx_hbm, tab_hbm, base, col0, idx_v, chunk_v, x_v, sem):
    pltpu.sync_copy(idx_hbm.at[pl.ds(base, _BPW)], idx_v)
    lane = lax.iota(jnp.int32, _L)

    def body(k, carry):
        copies = []
        for g in range(_CH // _L):
            off = k * _CH + g * _L
            i16 = idx_v[pl.ds(off, _L)]
            for l in range(_L):
                r = jnp.sum(jnp.where(lane == l, i16, 0))
                copies.append(pltpu.async_copy(
                    tab_hbm.at[pl.ds(r, 1), :],
                    chunk_v.at[pl.ds(g * _L + l, 1), :], sem))
        for c in copies:
            c.wait()
        for g in range(_CH // _L):
            src_rows = lax.iota(jnp.int32, _L) + jnp.int32(g * _L)
            dst_rows = src_rows + k * _CH
            for c in range(_D):
                val = plsc.load_gather(
                    chunk_v, [src_rows, jnp.full((_L,), c, jnp.int32)])
                plsc.store_scatter(
                    x_v, [dst_rows, jnp.full((_L,), col0 + c, jnp.int32)], val)
        return carry

    lax.fori_loop(0, _NCH, body, jnp.int32(0))


def _sc_gather_body(uidx_hbm, gidx_hbm, ptab_hbm, gtab_hbm,
                    x_out, idx_v, chunk_v, x_v, sem):
    wid = lax.axis_index("s") * 2 + lax.axis_index("c")
    base = wid * _BPW
    _gather_table(uidx_hbm, ptab_hbm, base, 0, idx_v, chunk_v, x_v, sem)
    _gather_table(gidx_hbm, gtab_hbm, base, _D, idx_v, chunk_v, x_v, sem)
    pltpu.sync_copy(x_v, x_out.at[pl.ds(base, _BPW), :])


@functools.cache
def _make_sc_gather():
    return functools.partial(
        pl.kernel,
        mesh=plsc.VectorSubcoreMesh(core_axis_name="c", subcore_axis_name="s"),
        compiler_params=pltpu.CompilerParams(needs_layout_passes=False),
        out_type=jax.ShapeDtypeStruct((_B, 2 * _D), jnp.float32),
        scratch_types=[
            pltpu.VMEM((_BPW,), jnp.int32),
            pltpu.VMEM((_CH, _D), jnp.float32),
            pltpu.VMEM((_BPW, 2 * _D), jnp.float32),
            pltpu.SemaphoreType.DMA,
        ],
    )(_sc_gather_body)


def _mlp_body(x_ref, w1_ref, b1_ref, w2_ref, b2_ref,
              w3_ref, b3_ref, w4_ref, b4_ref, o_ref):
    f32 = jnp.float32
    h = jnp.maximum(
        jnp.dot(x_ref[...], w1_ref[...], preferred_element_type=f32)
        + b1_ref[...], 0.0)
    h = jnp.maximum(
        jnp.dot(h, w2_ref[...], preferred_element_type=f32) + b2_ref[...], 0.0)
    h = jnp.maximum(
        jnp.dot(h, w3_ref[...], preferred_element_type=f32) + b3_ref[...], 0.0)
    o_ref[...] = jnp.dot(h, w4_ref[...], preferred_element_type=f32) + b4_ref[...]


def _mlp(x, w1, b1r, w2, b2r, w3, b3r, w4, b4r):
    full = lambda shape: pl.BlockSpec(shape, lambda i: (0, 0))
    return pl.pallas_call(
        _mlp_body,
        grid=(_B // _BS,),
        in_specs=[
            pl.BlockSpec((_BS, 2 * _D), lambda i: (i, 0)),
            full(w1.shape), full(b1r.shape),
            full(w2.shape), full(b2r.shape),
            full(w3.shape), full(b3r.shape),
            full(w4.shape), full(b4r.shape),
        ],
        out_specs=pl.BlockSpec((_BS, 1), lambda i: (i, 0)),
        out_shape=jax.ShapeDtypeStruct((_B, 1), jnp.float32),
    )(x, w1, b1r, w2, b2r, w3, b3r, w4, b4r)


def kernel(user, game, player_table, game_table, W1, b1, W2, b2, W3, b3, W4, b4):
    uidx = user.reshape(_B)
    gidx = game.reshape(_B)
    x = _make_sc_gather()(uidx, gidx, player_table, game_table)
    return _mlp(x, W1, b1.reshape(1, -1),
                W2, b2.reshape(1, -1),
                W3, b3.reshape(1, -1),
                W4, b4.reshape(1, 1))
